# Initial kernel scaffold; baseline (speedup 1.0000x reference)
import jax
import jax.numpy as jnp
from jax.experimental import pallas as pl


def kernel(x, edge_index, batch, W1, b1, W2, b2, W3, b3, A1, c1, A2, c2):
    raise NotImplementedError("write your pallas kernel here")


# speedup vs baseline: 8.4475x; 8.4475x over previous
import functools

import jax
import jax.numpy as jnp
from jax import lax
from jax.experimental import pallas as pl
from jax.experimental.pallas import tpu as pltpu
from jax.experimental.pallas import tpu_sc as plsc

_NSUB = 16
_CH = 80
_ZR = 16
_G = 16
_BN = 1000


def _hp_dot(a, b):
    return jnp.dot(a, b, precision=lax.Precision.HIGHEST,
                   preferred_element_type=jnp.float32)


@functools.lru_cache(None)
def _deg_kernel(n, e):
    rpt = (n // _NSUB) // 8 * 8
    tail = n - rpt * _NSUB
    assert rpt % _ZR == 0 and tail % _ZR == 0 and tail >= 0
    epw = e // (2 * _NSUB)
    nchunk = epw // _CH
    nz = rpt // _ZR
    mesh = plsc.VectorSubcoreMesh(core_axis_name="c", subcore_axis_name="s")

    @functools.partial(
        pl.kernel,
        out_type=[jax.ShapeDtypeStruct((n, 128), jnp.float32),
                  jax.ShapeDtypeStruct((n, 128), jnp.float32)],
        mesh=mesh,
        scratch_types=[
            pltpu.VMEM((_CH,), jnp.int32),
            pltpu.VMEM((_CH, 128), jnp.float32),
            pltpu.VMEM((_ZR, 128), jnp.float32),
            pltpu.VMEM_SHARED((n, 128), jnp.float32),
        ],
    )
    def deg(dst_hbm, deg0_hbm, deg1_hbm, dst_v, ones_v, zero_v, acc):
        c = lax.axis_index("c")
        s = lax.axis_index("s")

        @pl.loop(0, _ZR)
        def _(i):
            @pl.loop(0, 128, step=16)
            def _(j):
                zero_v[i, pl.ds(j, 16)] = jnp.zeros((16,), jnp.float32)

        @pl.loop(0, _CH)
        def _(i):
            @pl.loop(0, 128, step=16)
            def _(j):
                ones_v[i, pl.ds(j, 16)] = jnp.ones((16,), jnp.float32)

        base = s * rpt

        @pl.loop(0, nz)
        def _(k):
            pltpu.sync_copy(zero_v, acc.at[pl.ds(base + k * _ZR, _ZR)])

        if tail:
            @pl.when(s == _NSUB - 1)
            def _():
                @pl.loop(0, tail // _ZR)
                def _(k):
                    pltpu.sync_copy(
                        zero_v, acc.at[pl.ds(rpt * _NSUB + k * _ZR, _ZR)])

        plsc.subcore_barrier()

        ebase = (c * _NSUB + s) * epw

        @pl.loop(0, nchunk)
        def _(j):
            pltpu.sync_copy(dst_hbm.at[pl.ds(ebase + j * _CH, _CH)], dst_v)
            pltpu.sync_copy(ones_v, acc.at[dst_v], add=True)

        plsc.subcore_barrier()

        def flush(o_hbm):
            pltpu.sync_copy(acc.at[pl.ds(base, rpt)],
                            o_hbm.at[pl.ds(base, rpt)])
            if tail:
                @pl.when(s == _NSUB - 1)
                def _():
                    pltpu.sync_copy(acc.at[pl.ds(rpt * _NSUB, tail)],
                                    o_hbm.at[pl.ds(rpt * _NSUB, tail)])

        @pl.when(c == 0)
        def _():
            flush(deg0_hbm)

        @pl.when(c == 1)
        def _():
            flush(deg1_hbm)

    return deg


@functools.lru_cache(None)
def _prop_kernel(n, e, w):
    rpt = (n // _NSUB) // 8 * 8
    tail = n - rpt * _NSUB
    assert rpt % _ZR == 0 and tail % _ZR == 0 and tail >= 0
    epw = e // _NSUB
    nchunk = epw // _CH
    nz = rpt // _ZR
    mesh = plsc.VectorSubcoreMesh(core_axis_name="c", subcore_axis_name="s")

    @functools.partial(
        pl.kernel,
        out_type=[jax.ShapeDtypeStruct((n, w), jnp.float32),
                  jax.ShapeDtypeStruct((n, w), jnp.float32)],
        mesh=mesh,
        scratch_types=[
            pltpu.VMEM((_CH,), jnp.int32),
            pltpu.VMEM((_CH,), jnp.int32),
            pltpu.VMEM((_CH, w), jnp.float32),
            pltpu.VMEM((_ZR, w), jnp.float32),
            pltpu.VMEM_SHARED((n, w), jnp.float32),
        ],
    )
    def prop(src_hbm, dst_hbm, tlo_hbm, thi_hbm, olo_hbm, ohi_hbm,
             src_v, dst_v, rows_v, zero_v, acc):
        c = lax.axis_index("c")
        s = lax.axis_index("s")

        @pl.loop(0, _ZR)
        def _(i):
            @pl.loop(0, w, step=16)
            def _(j):
                zero_v[i, pl.ds(j, 16)] = jnp.zeros((16,), jnp.float32)

        base = s * rpt

        @pl.loop(0, nz)
        def _(k):
            pltpu.sync_copy(zero_v, acc.at[pl.ds(base + k * _ZR, _ZR)])

        if tail:
            @pl.when(s == _NSUB - 1)
            def _():
                @pl.loop(0, tail // _ZR)
                def _(k):
                    pltpu.sync_copy(
                        zero_v, acc.at[pl.ds(rpt * _NSUB + k * _ZR, _ZR)])

        plsc.subcore_barrier()

        ebase = s * epw

        def run(t_hbm, o_hbm):
            @pl.loop(0, nchunk)
            def _(j):
                off = ebase + j * _CH
                pltpu.sync_copy(src_hbm.at[pl.ds(off, _CH)], src_v)
                pltpu.sync_copy(dst_hbm.at[pl.ds(off, _CH)], dst_v)
                pltpu.sync_copy(t_hbm.at[src_v], rows_v)
                pltpu.sync_copy(rows_v, acc.at[dst_v], add=True)

            plsc.subcore_barrier()
            pltpu.sync_copy(acc.at[pl.ds(base, rpt)],
                            o_hbm.at[pl.ds(base, rpt)])
            if tail:
                @pl.when(s == _NSUB - 1)
                def _():
                    pltpu.sync_copy(acc.at[pl.ds(rpt * _NSUB, tail)],
                                    o_hbm.at[pl.ds(rpt * _NSUB, tail)])

        @pl.when(c == 0)
        def _():
            run(tlo_hbm, olo_hbm)

        @pl.when(c == 1)
        def _():
            run(thi_hbm, ohi_hbm)

    return prop


@functools.lru_cache(None)
def _prop_edge_split(n, e, w):
    rpt = (n // _NSUB) // 8 * 8
    tail = n - rpt * _NSUB
    assert rpt % _ZR == 0 and tail % _ZR == 0 and tail >= 0
    epw = e // (2 * _NSUB)
    nchunk = epw // _CH
    nz = rpt // _ZR
    mesh = plsc.VectorSubcoreMesh(core_axis_name="c", subcore_axis_name="s")

    @functools.partial(
        pl.kernel,
        out_type=[jax.ShapeDtypeStruct((n, w), jnp.float32),
                  jax.ShapeDtypeStruct((n, w), jnp.float32)],
        mesh=mesh,
        scratch_types=[
            pltpu.VMEM((_CH,), jnp.int32),
            pltpu.VMEM((_CH,), jnp.int32),
            pltpu.VMEM((_CH, w), jnp.float32),
            pltpu.VMEM((_ZR, w), jnp.float32),
            pltpu.VMEM_SHARED((n, w), jnp.float32),
        ],
    )
    def prop(src_hbm, dst_hbm, t_hbm, o0_hbm, o1_hbm,
             src_v, dst_v, rows_v, zero_v, acc):
        c = lax.axis_index("c")
        s = lax.axis_index("s")

        @pl.loop(0, _ZR)
        def _(i):
            @pl.loop(0, w, step=16)
            def _(j):
                zero_v[i, pl.ds(j, 16)] = jnp.zeros((16,), jnp.float32)

        base = s * rpt

        @pl.loop(0, nz)
        def _(k):
            pltpu.sync_copy(zero_v, acc.at[pl.ds(base + k * _ZR, _ZR)])

        if tail:
            @pl.when(s == _NSUB - 1)
            def _():
                @pl.loop(0, tail // _ZR)
                def _(k):
                    pltpu.sync_copy(
                        zero_v, acc.at[pl.ds(rpt * _NSUB + k * _ZR, _ZR)])

        plsc.subcore_barrier()

        ebase = (c * _NSUB + s) * epw

        @pl.loop(0, nchunk)
        def _(j):
            off = ebase + j * _CH
            pltpu.sync_copy(src_hbm.at[pl.ds(off, _CH)], src_v)
            pltpu.sync_copy(dst_hbm.at[pl.ds(off, _CH)], dst_v)
            pltpu.sync_copy(t_hbm.at[src_v], rows_v)
            pltpu.sync_copy(rows_v, acc.at[dst_v], add=True)

        plsc.subcore_barrier()

        def flush(o_hbm):
            pltpu.sync_copy(acc.at[pl.ds(base, rpt)],
                            o_hbm.at[pl.ds(base, rpt)])
            if tail:
                @pl.when(s == _NSUB - 1)
                def _():
                    pltpu.sync_copy(acc.at[pl.ds(rpt * _NSUB, tail)],
                                    o_hbm.at[pl.ds(rpt * _NSUB, tail)])

        @pl.when(c == 0)
        def _():
            flush(o0_hbm)

        @pl.when(c == 1)
        def _():
            flush(o1_hbm)

    return prop


@functools.lru_cache(None)
def _tc_first(n, f_in, h):
    hh = h // 2

    def body(x_ref, w_ref, d0_ref, d1_ref, lo_ref, hi_ref, dinv_ref):
        deg = d0_ref[...][:, 0:1] + d1_ref[...][:, 0:1] + 1.0
        dinv = lax.rsqrt(deg)
        t = _hp_dot(x_ref[...], w_ref[...])
        lo_ref[...] = t[:, :hh] * dinv
        hi_ref[...] = t[:, hh:] * dinv
        dinv_ref[...] = jnp.broadcast_to(dinv, (_BN, 16))

    return pl.pallas_call(
        body,
        grid=(n // _BN,),
        in_specs=[
            pl.BlockSpec((_BN, f_in), lambda i: (i, 0)),
            pl.BlockSpec((f_in, h), lambda i: (0, 0)),
            pl.BlockSpec((_BN, 128), lambda i: (i, 0)),
            pl.BlockSpec((_BN, 128), lambda i: (i, 0)),
        ],
        out_specs=[
            pl.BlockSpec((_BN, hh), lambda i: (i, 0)),
            pl.BlockSpec((_BN, hh), lambda i: (i, 0)),
            pl.BlockSpec((_BN, 16), lambda i: (i, 0)),
        ],
        out_shape=[jax.ShapeDtypeStruct((n, hh), jnp.float32),
                   jax.ShapeDtypeStruct((n, hh), jnp.float32),
                   jax.ShapeDtypeStruct((n, 16), jnp.float32)],
    )


@functools.lru_cache(None)
def _tc_mid(n, h_in, h_out, split_out):
    ih = h_in // 2
    oh = h_out // 2

    def body(alo_ref, ahi_ref, tlo_ref, thi_ref, b_ref, w_ref,
             dinv_ref, *out_refs):
        dinv = dinv_ref[...][:, 0:1]
        hcat = jnp.concatenate(
            [alo_ref[...] + tlo_ref[...], ahi_ref[...] + thi_ref[...]], axis=1)
        hact = jnp.maximum(hcat * dinv + b_ref[...], 0.0)
        t = _hp_dot(hact, w_ref[...])
        if split_out:
            out_refs[0][...] = t[:, :oh] * dinv
            out_refs[1][...] = t[:, oh:] * dinv
        else:
            out_refs[0][...] = t * dinv

    if split_out:
        out_specs = [pl.BlockSpec((_BN, oh), lambda i: (i, 0)),
                     pl.BlockSpec((_BN, oh), lambda i: (i, 0))]
        out_shape = [jax.ShapeDtypeStruct((n, oh), jnp.float32),
                     jax.ShapeDtypeStruct((n, oh), jnp.float32)]
    else:
        out_specs = [pl.BlockSpec((_BN, h_out), lambda i: (i, 0))]
        out_shape = [jax.ShapeDtypeStruct((n, h_out), jnp.float32)]

    return pl.pallas_call(
        body,
        grid=(n // _BN,),
        in_specs=[
            pl.BlockSpec((_BN, ih), lambda i: (i, 0)),
            pl.BlockSpec((_BN, ih), lambda i: (i, 0)),
            pl.BlockSpec((_BN, ih), lambda i: (i, 0)),
            pl.BlockSpec((_BN, ih), lambda i: (i, 0)),
            pl.BlockSpec((1, h_in), lambda i: (0, 0)),
            pl.BlockSpec((h_in, h_out), lambda i: (0, 0)),
            pl.BlockSpec((_BN, 16), lambda i: (i, 0)),
        ],
        out_specs=out_specs,
        out_shape=out_shape,
    )


@functools.lru_cache(None)
def _tc_last(n, h_out, mh):
    nblk = n // _BN

    def body(p0_ref, p1_ref, t_ref, b_ref, dinv_ref,
             bat_ref, a1_ref, c1_ref, a2_ref, c2_ref,
             h_ref, aout_ref, seg_ref, cnt_ref):
        i = pl.program_id(0)
        dinv = dinv_ref[...][:, 0:1]
        hcat = p0_ref[...] + p1_ref[...] + t_ref[...]
        hblk = hcat * dinv + b_ref[...]
        h_ref[...] = hblk

        bb = bat_ref[0, 0, :]
        onehot = (bb[:, None] ==
                  lax.broadcasted_iota(jnp.int32, (_BN, _G), 1)
                  ).astype(jnp.float32)
        seg_inc = lax.dot_general(onehot, hblk, (((0,), (0,)), ((), ())),
                                  precision=lax.Precision.HIGHEST,
                                  preferred_element_type=jnp.float32)
        cnt_inc = jnp.sum(onehot, axis=0)[None, :]

        @pl.when(i == 0)
        def _():
            seg_ref[...] = jnp.zeros_like(seg_ref)
            cnt_ref[...] = jnp.zeros_like(cnt_ref)

        seg_ref[...] += seg_inc
        cnt_ref[...] += cnt_inc

        @pl.when(i == nblk - 1)
        def _():
            cnt = jnp.maximum(cnt_ref[0, :], 1.0)
            gmean = seg_ref[...] / cnt[:, None]
            z = jnp.maximum(_hp_dot(gmean, a1_ref[...]) + c1_ref[...], 0.0)
            z2 = _hp_dot(z, a2_ref[...]) + c2_ref[...]
            aout_ref[...] = jax.nn.sigmoid(z2)

    return pl.pallas_call(
        body,
        grid=(nblk,),
        in_specs=[
            pl.BlockSpec((_BN, h_out), lambda i: (i, 0)),
            pl.BlockSpec((_BN, h_out), lambda i: (i, 0)),
            pl.BlockSpec((_BN, h_out), lambda i: (i, 0)),
            pl.BlockSpec((1, h_out), lambda i: (0, 0)),
            pl.BlockSpec((_BN, 16), lambda i: (i, 0)),
            pl.BlockSpec((1, 1, _BN), lambda i: (i, 0, 0)),
            pl.BlockSpec((h_out, mh), lambda i: (0, 0)),
            pl.BlockSpec((1, mh), lambda i: (0, 0)),
            pl.BlockSpec((mh, 1), lambda i: (0, 0)),
            pl.BlockSpec((1, 1), lambda i: (0, 0)),
        ],
        out_specs=[
            pl.BlockSpec((_BN, h_out), lambda i: (i, 0)),
            pl.BlockSpec((_G, 1), lambda i: (0, 0)),
        ],
        out_shape=[jax.ShapeDtypeStruct((n, h_out), jnp.float32),
                   jax.ShapeDtypeStruct((_G, 1), jnp.float32)],
        scratch_shapes=[pltpu.VMEM((_G, h_out), jnp.float32),
                        pltpu.VMEM((1, _G), jnp.float32)],
    )


def kernel(x, edge_index, batch, W1, b1, W2, b2, W3, b3, A1, c1, A2, c2):
    n, f_in = x.shape
    e = edge_index.shape[1]
    h = W1.shape[1]
    out = W3.shape[1]
    mh = A1.shape[1]

    src = edge_index[0]
    dst = edge_index[1]

    deg0, deg1 = _deg_kernel(n, e)(dst)

    t1lo, t1hi, dinv = _tc_first(n, f_in, h)(x, W1, deg0, deg1)
    a1lo, a1hi = _prop_kernel(n, e, h // 2)(src, dst, t1lo, t1hi)

    t2lo, t2hi = _tc_mid(n, h, h, True)(a1lo, a1hi, t1lo, t1hi,
                                        b1.reshape(1, h), W2, dinv)
    a2lo, a2hi = _prop_kernel(n, e, h // 2)(src, dst, t2lo, t2hi)

    (t3,) = _tc_mid(n, h, out, False)(a2lo, a2hi, t2lo, t2hi,
                                      b2.reshape(1, h), W3, dinv)
    a3p0, a3p1 = _prop_edge_split(n, e, out)(src, dst, t3)

    hfinal, a = _tc_last(n, out, mh)(
        a3p0, a3p1, t3, b3.reshape(1, out), dinv,
        batch.reshape(n // _BN, 1, _BN),
        A1, c1.reshape(1, mh), A2, c2.reshape(1, 1))
    return (hfinal, a)


# speedup vs baseline: 15.9805x; 1.8918x over previous
import functools

import jax
import jax.numpy as jnp
from jax import lax
from jax.experimental import pallas as pl
from jax.experimental.pallas import tpu as pltpu
from jax.experimental.pallas import tpu_sc as plsc

_NSUB = 16
_CH = 80
_ZR = 16
_G = 16
_BN = 1000


def _hp_dot(a, b):
    return jnp.dot(a, b, precision=lax.Precision.HIGHEST,
                   preferred_element_type=jnp.float32)


def _zero_acc(acc, zero_v, s, base, rpt, tail, w):
    @pl.loop(0, _ZR)
    def _(i):
        @pl.loop(0, w, step=16)
        def _(j):
            zero_v[i, pl.ds(j, 16)] = jnp.zeros((16,), jnp.float32)

    @pl.loop(0, rpt // _ZR)
    def _(k):
        pltpu.sync_copy(zero_v, acc.at[pl.ds(base + k * _ZR, _ZR)])

    if tail:
        @pl.when(s == _NSUB - 1)
        def _():
            @pl.loop(0, tail // _ZR)
            def _(k):
                pltpu.sync_copy(zero_v,
                                acc.at[pl.ds(rpt * _NSUB + k * _ZR, _ZR)])


def _flush_acc(acc, o_hbm, s, base, rpt, tail):
    pltpu.sync_copy(acc.at[pl.ds(base, rpt)], o_hbm.at[pl.ds(base, rpt)])
    if tail:
        @pl.when(s == _NSUB - 1)
        def _():
            pltpu.sync_copy(acc.at[pl.ds(rpt * _NSUB, tail)],
                            o_hbm.at[pl.ds(rpt * _NSUB, tail)])


@functools.lru_cache(None)
def _deg_kernel(n, e):
    rpt = (n // _NSUB) // 8 * 8
    tail = n - rpt * _NSUB
    assert rpt % _ZR == 0 and tail % _ZR == 0 and tail >= 0
    epw = e // (2 * _NSUB)
    nchunk = epw // _CH
    last = nchunk - 1
    mesh = plsc.VectorSubcoreMesh(core_axis_name="c", subcore_axis_name="s")

    @functools.partial(
        pl.kernel,
        out_type=[jax.ShapeDtypeStruct((n, 128), jnp.float32),
                  jax.ShapeDtypeStruct((n, 128), jnp.float32)],
        mesh=mesh,
        scratch_types=[
            pltpu.VMEM((2, _CH), jnp.int32),
            pltpu.VMEM((_CH, 128), jnp.float32),
            pltpu.VMEM((_ZR, 128), jnp.float32),
            pltpu.VMEM_SHARED((n, 128), jnp.float32),
            pltpu.SemaphoreType.DMA,
            pltpu.SemaphoreType.DMA,
        ],
    )
    def deg(dst_hbm, deg0_hbm, deg1_hbm, dst_v, ones_v, zero_v, acc,
            semi0, semi1):
        c = lax.axis_index("c")
        s = lax.axis_index("s")

        @pl.loop(0, _CH)
        def _(i):
            @pl.loop(0, 128, step=16)
            def _(j):
                ones_v[i, pl.ds(j, 16)] = jnp.ones((16,), jnp.float32)

        base = s * rpt
        _zero_acc(acc, zero_v, s, base, rpt, tail, 128)
        plsc.subcore_barrier()

        ebase = (c * _NSUB + s) * epw

        def i_start(j, b, sem):
            pltpu.make_async_copy(dst_hbm.at[pl.ds(ebase + j * _CH, _CH)],
                                  dst_v.at[b], sem).start()

        def i_wait(b, sem):
            pltpu.make_async_copy(dst_hbm.at[pl.ds(ebase, _CH)],
                                  dst_v.at[b], sem).wait()

        def sc_add(b):
            pltpu.sync_copy(ones_v, acc.at[dst_v.at[b]], add=True)

        pltpu.sync_copy(dst_hbm.at[pl.ds(ebase, _CH)], dst_v.at[0])
        i_start(1, 1, semi1)

        @pl.loop(0, nchunk // 2)
        def _(jj):
            j0 = 2 * jj
            j2c = jnp.minimum(j0 + 2, last)
            j3c = jnp.minimum(j0 + 3, last)
            sc_add(0)
            i_start(j2c, 0, semi0)
            i_wait(1, semi1)
            sc_add(1)
            i_start(j3c, 1, semi1)
            i_wait(0, semi0)

        i_wait(1, semi1)
        if nchunk % 2:
            sc_add(0)

        plsc.subcore_barrier()

        @pl.when(c == 0)
        def _():
            _flush_acc(acc, deg0_hbm, s, base, rpt, tail)

        @pl.when(c == 1)
        def _():
            _flush_acc(acc, deg1_hbm, s, base, rpt, tail)

    return deg


def _edge_pipeline(src_hbm, dst_hbm, t_hbm, acc, ebase, nchunk,
                   src_v, dst_v, rows0_v, rows1_v,
                   semg0, semg1, semi0, semi1):
    last = nchunk - 1

    def i_start(j, b, sem):
        pltpu.make_async_copy(src_hbm.at[pl.ds(ebase + j * _CH, _CH)],
                              src_v.at[b], sem).start()
        pltpu.make_async_copy(dst_hbm.at[pl.ds(ebase + j * _CH, _CH)],
                              dst_v.at[b], sem).start()

    def i_wait(b, sem):
        pltpu.make_async_copy(src_hbm.at[pl.ds(ebase, _CH)],
                              src_v.at[b], sem).wait()
        pltpu.make_async_copy(dst_hbm.at[pl.ds(ebase, _CH)],
                              dst_v.at[b], sem).wait()

    def g_start(b, rows, sem):
        pltpu.make_async_copy(t_hbm.at[src_v.at[b]], rows, sem).start()

    def g_wait(b, rows, sem):
        pltpu.make_async_copy(t_hbm.at[src_v.at[b]], rows, sem).wait()

    def sc_add(b, rows):
        pltpu.sync_copy(rows, acc.at[dst_v.at[b]], add=True)

    pltpu.sync_copy(src_hbm.at[pl.ds(ebase, _CH)], src_v.at[0])
    pltpu.sync_copy(dst_hbm.at[pl.ds(ebase, _CH)], dst_v.at[0])
    g_start(0, rows0_v, semg0)
    i_start(1, 1, semi1)

    @pl.loop(0, nchunk // 2)
    def _(jj):
        j0 = 2 * jj
        j2c = jnp.minimum(j0 + 2, last)
        j3c = jnp.minimum(j0 + 3, last)
        i_wait(1, semi1)
        g_wait(0, rows0_v, semg0)
        g_start(1, rows1_v, semg1)
        sc_add(0, rows0_v)
        i_start(j2c, 0, semi0)
        g_wait(1, rows1_v, semg1)
        i_wait(0, semi0)
        g_start(0, rows0_v, semg0)
        sc_add(1, rows1_v)
        i_start(j3c, 1, semi1)

    i_wait(1, semi1)
    g_wait(0, rows0_v, semg0)
    if nchunk % 2:
        sc_add(0, rows0_v)


@functools.lru_cache(None)
def _prop_kernel(n, e, w):
    rpt = (n // _NSUB) // 8 * 8
    tail = n - rpt * _NSUB
    assert rpt % _ZR == 0 and tail % _ZR == 0 and tail >= 0
    epw = e // _NSUB
    nchunk = epw // _CH
    mesh = plsc.VectorSubcoreMesh(core_axis_name="c", subcore_axis_name="s")

    @functools.partial(
        pl.kernel,
        out_type=[jax.ShapeDtypeStruct((n, w), jnp.float32),
                  jax.ShapeDtypeStruct((n, w), jnp.float32)],
        mesh=mesh,
        scratch_types=[
            pltpu.VMEM((2, _CH), jnp.int32),
            pltpu.VMEM((2, _CH), jnp.int32),
            pltpu.VMEM((_CH, w), jnp.float32),
            pltpu.VMEM((_CH, w), jnp.float32),
            pltpu.VMEM((_ZR, w), jnp.float32),
            pltpu.VMEM_SHARED((n, w), jnp.float32),
            pltpu.SemaphoreType.DMA,
            pltpu.SemaphoreType.DMA,
            pltpu.SemaphoreType.DMA,
            pltpu.SemaphoreType.DMA,
        ],
    )
    def prop(src_hbm, dst_hbm, tlo_hbm, thi_hbm, olo_hbm, ohi_hbm,
             src_v, dst_v, rows0_v, rows1_v, zero_v, acc,
             semg0, semg1, semi0, semi1):
        c = lax.axis_index("c")
        s = lax.axis_index("s")

        base = s * rpt
        _zero_acc(acc, zero_v, s, base, rpt, tail, w)
        plsc.subcore_barrier()

        ebase = s * epw

        def run(t_hbm, o_hbm):
            _edge_pipeline(src_hbm, dst_hbm, t_hbm, acc, ebase, nchunk,
                           src_v, dst_v, rows0_v, rows1_v,
                           semg0, semg1, semi0, semi1)
            plsc.subcore_barrier()
            _flush_acc(acc, o_hbm, s, base, rpt, tail)

        @pl.when(c == 0)
        def _():
            run(tlo_hbm, olo_hbm)

        @pl.when(c == 1)
        def _():
            run(thi_hbm, ohi_hbm)

    return prop


@functools.lru_cache(None)
def _prop_edge_split(n, e, w):
    rpt = (n // _NSUB) // 8 * 8
    tail = n - rpt * _NSUB
    assert rpt % _ZR == 0 and tail % _ZR == 0 and tail >= 0
    epw = e // (2 * _NSUB)
    nchunk = epw // _CH
    mesh = plsc.VectorSubcoreMesh(core_axis_name="c", subcore_axis_name="s")

    @functools.partial(
        pl.kernel,
        out_type=[jax.ShapeDtypeStruct((n, w), jnp.float32),
                  jax.ShapeDtypeStruct((n, w), jnp.float32)],
        mesh=mesh,
        scratch_types=[
            pltpu.VMEM((2, _CH), jnp.int32),
            pltpu.VMEM((2, _CH), jnp.int32),
            pltpu.VMEM((_CH, w), jnp.float32),
            pltpu.VMEM((_CH, w), jnp.float32),
            pltpu.VMEM((_ZR, w), jnp.float32),
            pltpu.VMEM_SHARED((n, w), jnp.float32),
            pltpu.SemaphoreType.DMA,
            pltpu.SemaphoreType.DMA,
            pltpu.SemaphoreType.DMA,
            pltpu.SemaphoreType.DMA,
        ],
    )
    def prop(src_hbm, dst_hbm, t_hbm, o0_hbm, o1_hbm,
             src_v, dst_v, rows0_v, rows1_v, zero_v, acc,
             semg0, semg1, semi0, semi1):
        c = lax.axis_index("c")
        s = lax.axis_index("s")

        base = s * rpt
        _zero_acc(acc, zero_v, s, base, rpt, tail, w)
        plsc.subcore_barrier()

        ebase = (c * _NSUB + s) * epw

        _edge_pipeline(src_hbm, dst_hbm, t_hbm, acc, ebase, nchunk,
                       src_v, dst_v, rows0_v, rows1_v,
                       semg0, semg1, semi0, semi1)
        plsc.subcore_barrier()

        @pl.when(c == 0)
        def _():
            _flush_acc(acc, o0_hbm, s, base, rpt, tail)

        @pl.when(c == 1)
        def _():
            _flush_acc(acc, o1_hbm, s, base, rpt, tail)

    return prop


@functools.lru_cache(None)
def _tc_first(n, f_in, h):
    hh = h // 2

    def body(x_ref, w_ref, d0_ref, d1_ref, lo_ref, hi_ref, dinv_ref):
        deg = d0_ref[...][:, 0:1] + d1_ref[...][:, 0:1] + 1.0
        dinv = lax.rsqrt(deg)
        t = _hp_dot(x_ref[...], w_ref[...])
        lo_ref[...] = t[:, :hh] * dinv
        hi_ref[...] = t[:, hh:] * dinv
        dinv_ref[...] = jnp.broadcast_to(dinv, (_BN, 16))

    return pl.pallas_call(
        body,
        grid=(n // _BN,),
        in_specs=[
            pl.BlockSpec((_BN, f_in), lambda i: (i, 0)),
            pl.BlockSpec((f_in, h), lambda i: (0, 0)),
            pl.BlockSpec((_BN, 128), lambda i: (i, 0)),
            pl.BlockSpec((_BN, 128), lambda i: (i, 0)),
        ],
        out_specs=[
            pl.BlockSpec((_BN, hh), lambda i: (i, 0)),
            pl.BlockSpec((_BN, hh), lambda i: (i, 0)),
            pl.BlockSpec((_BN, 16), lambda i: (i, 0)),
        ],
        out_shape=[jax.ShapeDtypeStruct((n, hh), jnp.float32),
                   jax.ShapeDtypeStruct((n, hh), jnp.float32),
                   jax.ShapeDtypeStruct((n, 16), jnp.float32)],
    )


@functools.lru_cache(None)
def _tc_mid(n, h_in, h_out, split_out):
    ih = h_in // 2
    oh = h_out // 2

    def body(alo_ref, ahi_ref, tlo_ref, thi_ref, b_ref, w_ref,
             dinv_ref, *out_refs):
        dinv = dinv_ref[...][:, 0:1]
        hcat = jnp.concatenate(
            [alo_ref[...] + tlo_ref[...], ahi_ref[...] + thi_ref[...]], axis=1)
        hact = jnp.maximum(hcat * dinv + b_ref[...], 0.0)
        t = _hp_dot(hact, w_ref[...])
        if split_out:
            out_refs[0][...] = t[:, :oh] * dinv
            out_refs[1][...] = t[:, oh:] * dinv
        else:
            out_refs[0][...] = t * dinv

    if split_out:
        out_specs = [pl.BlockSpec((_BN, oh), lambda i: (i, 0)),
                     pl.BlockSpec((_BN, oh), lambda i: (i, 0))]
        out_shape = [jax.ShapeDtypeStruct((n, oh), jnp.float32),
                     jax.ShapeDtypeStruct((n, oh), jnp.float32)]
    else:
        out_specs = [pl.BlockSpec((_BN, h_out), lambda i: (i, 0))]
        out_shape = [jax.ShapeDtypeStruct((n, h_out), jnp.float32)]

    return pl.pallas_call(
        body,
        grid=(n // _BN,),
        in_specs=[
            pl.BlockSpec((_BN, ih), lambda i: (i, 0)),
            pl.BlockSpec((_BN, ih), lambda i: (i, 0)),
            pl.BlockSpec((_BN, ih), lambda i: (i, 0)),
            pl.BlockSpec((_BN, ih), lambda i: (i, 0)),
            pl.BlockSpec((1, h_in), lambda i: (0, 0)),
            pl.BlockSpec((h_in, h_out), lambda i: (0, 0)),
            pl.BlockSpec((_BN, 16), lambda i: (i, 0)),
        ],
        out_specs=out_specs,
        out_shape=out_shape,
    )


@functools.lru_cache(None)
def _tc_last(n, h_out, mh):
    nblk = n // _BN

    def body(p0_ref, p1_ref, t_ref, b_ref, dinv_ref,
             bat_ref, a1_ref, c1_ref, a2_ref, c2_ref,
             h_ref, aout_ref, seg_ref, cnt_ref):
        i = pl.program_id(0)
        dinv = dinv_ref[...][:, 0:1]
        hcat = p0_ref[...] + p1_ref[...] + t_ref[...]
        hblk = hcat * dinv + b_ref[...]
        h_ref[...] = hblk

        bb = bat_ref[0, 0, :]
        onehot = (bb[:, None] ==
                  lax.broadcasted_iota(jnp.int32, (_BN, _G), 1)
                  ).astype(jnp.float32)
        seg_inc = lax.dot_general(onehot, hblk, (((0,), (0,)), ((), ())),
                                  precision=lax.Precision.HIGHEST,
                                  preferred_element_type=jnp.float32)
        cnt_inc = jnp.sum(onehot, axis=0)[None, :]

        @pl.when(i == 0)
        def _():
            seg_ref[...] = jnp.zeros_like(seg_ref)
            cnt_ref[...] = jnp.zeros_like(cnt_ref)

        seg_ref[...] += seg_inc
        cnt_ref[...] += cnt_inc

        @pl.when(i == nblk - 1)
        def _():
            cnt = jnp.maximum(cnt_ref[0, :], 1.0)
            gmean = seg_ref[...] / cnt[:, None]
            z = jnp.maximum(_hp_dot(gmean, a1_ref[...]) + c1_ref[...], 0.0)
            z2 = _hp_dot(z, a2_ref[...]) + c2_ref[...]
            aout_ref[...] = jax.nn.sigmoid(z2)

    return pl.pallas_call(
        body,
        grid=(nblk,),
        in_specs=[
            pl.BlockSpec((_BN, h_out), lambda i: (i, 0)),
            pl.BlockSpec((_BN, h_out), lambda i: (i, 0)),
            pl.BlockSpec((_BN, h_out), lambda i: (i, 0)),
            pl.BlockSpec((1, h_out), lambda i: (0, 0)),
            pl.BlockSpec((_BN, 16), lambda i: (i, 0)),
            pl.BlockSpec((1, 1, _BN), lambda i: (i, 0, 0)),
            pl.BlockSpec((h_out, mh), lambda i: (0, 0)),
            pl.BlockSpec((1, mh), lambda i: (0, 0)),
            pl.BlockSpec((mh, 1), lambda i: (0, 0)),
            pl.BlockSpec((1, 1), lambda i: (0, 0)),
        ],
        out_specs=[
            pl.BlockSpec((_BN, h_out), lambda i: (i, 0)),
            pl.BlockSpec((_G, 1), lambda i: (0, 0)),
        ],
        out_shape=[jax.ShapeDtypeStruct((n, h_out), jnp.float32),
                   jax.ShapeDtypeStruct((_G, 1), jnp.float32)],
        scratch_shapes=[pltpu.VMEM((_G, h_out), jnp.float32),
                        pltpu.VMEM((1, _G), jnp.float32)],
    )


def kernel(x, edge_index, batch, W1, b1, W2, b2, W3, b3, A1, c1, A2, c2):
    n, f_in = x.shape
    e = edge_index.shape[1]
    h = W1.shape[1]
    out = W3.shape[1]
    mh = A1.shape[1]

    src = edge_index[0]
    dst = edge_index[1]

    deg0, deg1 = _deg_kernel(n, e)(dst)

    t1lo, t1hi, dinv = _tc_first(n, f_in, h)(x, W1, deg0, deg1)
    a1lo, a1hi = _prop_kernel(n, e, h // 2)(src, dst, t1lo, t1hi)

    t2lo, t2hi = _tc_mid(n, h, h, True)(a1lo, a1hi, t1lo, t1hi,
                                        b1.reshape(1, h), W2, dinv)
    a2lo, a2hi = _prop_kernel(n, e, h // 2)(src, dst, t2lo, t2hi)

    (t3,) = _tc_mid(n, h, out, False)(a2lo, a2hi, t2lo, t2hi,
                                      b2.reshape(1, h), W3, dinv)
    a3p0, a3p1 = _prop_edge_split(n, e, out)(src, dst, t3)

    hfinal, a = _tc_last(n, out, mh)(
        a3p0, a3p1, t3, b3.reshape(1, out), dinv,
        batch.reshape(n // _BN, 1, _BN),
        A1, c1.reshape(1, mh), A2, c2.reshape(1, 1))
    return (hfinal, a)


# speedup vs baseline: 23.1854x; 1.4508x over previous
import functools

import jax
import jax.numpy as jnp
from jax import lax
from jax.experimental import pallas as pl
from jax.experimental.pallas import tpu as pltpu
from jax.experimental.pallas import tpu_sc as plsc

_NSUB = 16
_CH = 80
_ZR = 16
_G = 16
_BN = 1000


def _hp_dot(a, b):
    return jnp.dot(a, b, precision=lax.Precision.HIGHEST,
                   preferred_element_type=jnp.float32)


def _zero_acc(acc, zero_v, s, base, rpt, tail, w):
    @pl.loop(0, _ZR)
    def _(i):
        @pl.loop(0, w, step=16)
        def _(j):
            zero_v[i, pl.ds(j, 16)] = jnp.zeros((16,), jnp.float32)

    @pl.loop(0, rpt // _ZR)
    def _(k):
        pltpu.sync_copy(zero_v, acc.at[pl.ds(base + k * _ZR, _ZR)])

    if tail:
        @pl.when(s == _NSUB - 1)
        def _():
            @pl.loop(0, tail // _ZR)
            def _(k):
                pltpu.sync_copy(zero_v,
                                acc.at[pl.ds(rpt * _NSUB + k * _ZR, _ZR)])


def _flush_acc(acc, o_hbm, s, base, rpt, tail):
    pltpu.sync_copy(acc.at[pl.ds(base, rpt)], o_hbm.at[pl.ds(base, rpt)])
    if tail:
        @pl.when(s == _NSUB - 1)
        def _():
            pltpu.sync_copy(acc.at[pl.ds(rpt * _NSUB, tail)],
                            o_hbm.at[pl.ds(rpt * _NSUB, tail)])


@functools.lru_cache(None)
def _deg_kernel(n, e):
    rpt = (n // _NSUB) // 8 * 8
    tail = n - rpt * _NSUB
    assert rpt % _ZR == 0 and tail % _ZR == 0 and tail >= 0
    epw = e // (2 * _NSUB)
    nchunk = epw // _CH
    last = nchunk - 1
    mesh = plsc.VectorSubcoreMesh(core_axis_name="c", subcore_axis_name="s")

    @functools.partial(
        pl.kernel,
        out_type=[jax.ShapeDtypeStruct((n, 128), jnp.float32),
                  jax.ShapeDtypeStruct((n, 128), jnp.float32)],
        mesh=mesh,
        scratch_types=[
            pltpu.VMEM((4, _CH), jnp.int32),
            pltpu.VMEM((_CH, 128), jnp.float32),
            pltpu.VMEM((_ZR, 128), jnp.float32),
            pltpu.VMEM_SHARED((n, 128), jnp.float32),
        ] + [pltpu.SemaphoreType.DMA] * 8,
    )
    def deg(dst_hbm, deg0_hbm, deg1_hbm, dst_v, ones_v, zero_v, acc, *sems):
        c = lax.axis_index("c")
        s = lax.axis_index("s")
        ss, si = sems[0:4], sems[4:8]

        @pl.loop(0, _CH)
        def _(i):
            @pl.loop(0, 128, step=16)
            def _(j):
                ones_v[i, pl.ds(j, 16)] = jnp.ones((16,), jnp.float32)

        base = s * rpt
        _zero_acc(acc, zero_v, s, base, rpt, tail, 128)
        plsc.subcore_barrier()

        ebase = (c * _NSUB + s) * epw

        def i_start(j, b):
            pltpu.make_async_copy(dst_hbm.at[pl.ds(ebase + j * _CH, _CH)],
                                  dst_v.at[b], si[b]).start()

        def i_wait(b):
            pltpu.make_async_copy(dst_hbm.at[pl.ds(ebase, _CH)],
                                  dst_v.at[b], si[b]).wait()

        def s_start(b):
            pltpu.async_copy(ones_v, acc.at[dst_v.at[b]], ss[b], add=True)

        def s_wait(b):
            pltpu.make_async_copy(ones_v, acc.at[dst_v.at[b]], ss[b]).wait()

        def make_slot(k):
            b, b2 = k % 4, (k + 2) % 4

            def run_slot(j, jp):
                i_wait(b)
                s_start(b)
                s_wait(b2)
                i_start(jp, b2)
            return run_slot

        slots = [make_slot(k) for k in range(4)]

        pltpu.sync_copy(dst_hbm.at[pl.ds(ebase, _CH)], dst_v.at[0])
        pltpu.sync_copy(dst_hbm.at[pl.ds(ebase + _CH, _CH)], dst_v.at[1])
        i_start(2, 2)
        i_start(3, 3)
        s_start(0)
        s_start(1)

        n_slots = nchunk - 2
        n_iter = n_slots // 4

        @pl.loop(0, n_iter)
        def _(jj):
            j0 = 2 + 4 * jj
            for t in range(4):
                slots[(2 + t) % 4](j0 + t, jnp.minimum(j0 + t + 2, last))

        for t in range(n_slots - 4 * n_iter):
            j = 2 + 4 * n_iter + t
            slots[j % 4](j, min(j + 2, last))

        i_wait((last + 1) % 4)
        i_wait((last + 2) % 4)
        s_wait((last - 1) % 4)
        s_wait(last % 4)

        plsc.subcore_barrier()

        @pl.when(c == 0)
        def _():
            _flush_acc(acc, deg0_hbm, s, base, rpt, tail)

        @pl.when(c == 1)
        def _():
            _flush_acc(acc, deg1_hbm, s, base, rpt, tail)

    return deg


def _edge_pipeline(src_hbm, dst_hbm, t_hbm, acc, ebase, nchunk,
                   src_v, dst_v, rows, sg, ss, si):
    last = nchunk - 1
    assert nchunk >= 4

    def i_start(j, b, sem):
        pltpu.make_async_copy(src_hbm.at[pl.ds(ebase + j * _CH, _CH)],
                              src_v.at[b], sem).start()
        pltpu.make_async_copy(dst_hbm.at[pl.ds(ebase + j * _CH, _CH)],
                              dst_v.at[b], sem).start()

    def i_sync(j, b):
        pltpu.sync_copy(src_hbm.at[pl.ds(ebase + j * _CH, _CH)], src_v.at[b])
        pltpu.sync_copy(dst_hbm.at[pl.ds(ebase + j * _CH, _CH)], dst_v.at[b])

    def i_wait(b):
        pltpu.make_async_copy(src_hbm.at[pl.ds(ebase, _CH)],
                              src_v.at[b], si[b]).wait()
        pltpu.make_async_copy(dst_hbm.at[pl.ds(ebase, _CH)],
                              dst_v.at[b], si[b]).wait()

    def g_start(b):
        pltpu.make_async_copy(t_hbm.at[src_v.at[b]], rows[b], sg[b]).start()

    def g_wait(b):
        pltpu.make_async_copy(t_hbm.at[src_v.at[b]], rows[b], sg[b]).wait()

    def s_start(b):
        pltpu.async_copy(rows[b], acc.at[dst_v.at[b]], ss[b], add=True)

    def s_wait(b):
        pltpu.make_async_copy(rows[b], acc.at[dst_v.at[b]], ss[b]).wait()

    def make_slot(k):
        b, b1, b2 = k % 4, (k - 1) % 4, (k + 2) % 4

        def run_slot(j, jp):
            i_wait(b)
            g_start(b)
            g_wait(b1)
            s_start(b1)
            s_wait(b2)
            i_start(jp, b2, si[b2])
        return run_slot

    slots = [make_slot(k) for k in range(4)]

    i_sync(0, 0)
    i_sync(1, 1)
    i_start(2, 2, si[2])
    i_start(3, 3, si[3])
    g_start(0)
    g_start(1)
    g_wait(0)
    s_start(0)

    n_slots = nchunk - 2
    n_iter = n_slots // 4

    @pl.loop(0, n_iter)
    def _(jj):
        j0 = 2 + 4 * jj
        for t in range(4):
            slots[(2 + t) % 4](j0 + t, jnp.minimum(j0 + t + 2, last))

    for t in range(n_slots - 4 * n_iter):
        j = 2 + 4 * n_iter + t
        slots[j % 4](j, min(j + 2, last))

    bL = last % 4
    g_wait(bL)
    s_start(bL)
    i_wait((last + 1) % 4)
    i_wait((last + 2) % 4)
    s_wait((last - 1) % 4)
    s_wait(bL)


@functools.lru_cache(None)
def _prop_kernel(n, e, w):
    rpt = (n // _NSUB) // 8 * 8
    tail = n - rpt * _NSUB
    assert rpt % _ZR == 0 and tail % _ZR == 0 and tail >= 0
    epw = e // _NSUB
    nchunk = epw // _CH
    mesh = plsc.VectorSubcoreMesh(core_axis_name="c", subcore_axis_name="s")

    @functools.partial(
        pl.kernel,
        out_type=[jax.ShapeDtypeStruct((n, w), jnp.float32),
                  jax.ShapeDtypeStruct((n, w), jnp.float32)],
        mesh=mesh,
        scratch_types=[
            pltpu.VMEM((4, _CH), jnp.int32),
            pltpu.VMEM((4, _CH), jnp.int32),
            pltpu.VMEM((_CH, w), jnp.float32),
            pltpu.VMEM((_CH, w), jnp.float32),
            pltpu.VMEM((_CH, w), jnp.float32),
            pltpu.VMEM((_CH, w), jnp.float32),
            pltpu.VMEM((_ZR, w), jnp.float32),
            pltpu.VMEM_SHARED((n, w), jnp.float32),
        ] + [pltpu.SemaphoreType.DMA] * 12,
    )
    def prop(src_hbm, dst_hbm, tlo_hbm, thi_hbm, olo_hbm, ohi_hbm,
             src_v, dst_v, r0, r1, r2, r3, zero_v, acc, *sems):
        c = lax.axis_index("c")
        s = lax.axis_index("s")
        rows, sg, ss, si = [r0, r1, r2, r3], sems[0:4], sems[4:8], sems[8:12]

        base = s * rpt
        _zero_acc(acc, zero_v, s, base, rpt, tail, w)
        plsc.subcore_barrier()

        ebase = s * epw

        def run(t_hbm, o_hbm):
            _edge_pipeline(src_hbm, dst_hbm, t_hbm, acc, ebase, nchunk,
                           src_v, dst_v, rows, sg, ss, si)
            plsc.subcore_barrier()
            _flush_acc(acc, o_hbm, s, base, rpt, tail)

        @pl.when(c == 0)
        def _():
            run(tlo_hbm, olo_hbm)

        @pl.when(c == 1)
        def _():
            run(thi_hbm, ohi_hbm)

    return prop


@functools.lru_cache(None)
def _prop_edge_split(n, e, w):
    rpt = (n // _NSUB) // 8 * 8
    tail = n - rpt * _NSUB
    assert rpt % _ZR == 0 and tail % _ZR == 0 and tail >= 0
    epw = e // (2 * _NSUB)
    nchunk = epw // _CH
    mesh = plsc.VectorSubcoreMesh(core_axis_name="c", subcore_axis_name="s")

    @functools.partial(
        pl.kernel,
        out_type=[jax.ShapeDtypeStruct((n, w), jnp.float32),
                  jax.ShapeDtypeStruct((n, w), jnp.float32)],
        mesh=mesh,
        scratch_types=[
            pltpu.VMEM((4, _CH), jnp.int32),
            pltpu.VMEM((4, _CH), jnp.int32),
            pltpu.VMEM((_CH, w), jnp.float32),
            pltpu.VMEM((_CH, w), jnp.float32),
            pltpu.VMEM((_CH, w), jnp.float32),
            pltpu.VMEM((_CH, w), jnp.float32),
            pltpu.VMEM((_ZR, w), jnp.float32),
            pltpu.VMEM_SHARED((n, w), jnp.float32),
        ] + [pltpu.SemaphoreType.DMA] * 12,
    )
    def prop(src_hbm, dst_hbm, t_hbm, o0_hbm, o1_hbm,
             src_v, dst_v, r0, r1, r2, r3, zero_v, acc, *sems):
        c = lax.axis_index("c")
        s = lax.axis_index("s")
        rows, sg, ss, si = [r0, r1, r2, r3], sems[0:4], sems[4:8], sems[8:12]

        base = s * rpt
        _zero_acc(acc, zero_v, s, base, rpt, tail, w)
        plsc.subcore_barrier()

        ebase = (c * _NSUB + s) * epw

        _edge_pipeline(src_hbm, dst_hbm, t_hbm, acc, ebase, nchunk,
                       src_v, dst_v, rows, sg, ss, si)
        plsc.subcore_barrier()

        @pl.when(c == 0)
        def _():
            _flush_acc(acc, o0_hbm, s, base, rpt, tail)

        @pl.when(c == 1)
        def _():
            _flush_acc(acc, o1_hbm, s, base, rpt, tail)

    return prop


@functools.lru_cache(None)
def _tc_first(n, f_in, h):
    hh = h // 2

    def body(x_ref, w_ref, d0_ref, d1_ref, lo_ref, hi_ref, dinv_ref):
        deg = d0_ref[...][:, 0:1] + d1_ref[...][:, 0:1] + 1.0
        dinv = lax.rsqrt(deg)
        t = _hp_dot(x_ref[...], w_ref[...])
        lo_ref[...] = t[:, :hh] * dinv
        hi_ref[...] = t[:, hh:] * dinv
        dinv_ref[...] = jnp.broadcast_to(dinv, (_BN, 16))

    return pl.pallas_call(
        body,
        grid=(n // _BN,),
        in_specs=[
            pl.BlockSpec((_BN, f_in), lambda i: (i, 0)),
            pl.BlockSpec((f_in, h), lambda i: (0, 0)),
            pl.BlockSpec((_BN, 128), lambda i: (i, 0)),
            pl.BlockSpec((_BN, 128), lambda i: (i, 0)),
        ],
        out_specs=[
            pl.BlockSpec((_BN, hh), lambda i: (i, 0)),
            pl.BlockSpec((_BN, hh), lambda i: (i, 0)),
            pl.BlockSpec((_BN, 16), lambda i: (i, 0)),
        ],
        out_shape=[jax.ShapeDtypeStruct((n, hh), jnp.float32),
                   jax.ShapeDtypeStruct((n, hh), jnp.float32),
                   jax.ShapeDtypeStruct((n, 16), jnp.float32)],
    )


@functools.lru_cache(None)
def _tc_mid(n, h_in, h_out, split_out):
    ih = h_in // 2
    oh = h_out // 2

    def body(alo_ref, ahi_ref, tlo_ref, thi_ref, b_ref, w_ref,
             dinv_ref, *out_refs):
        dinv = dinv_ref[...][:, 0:1]
        hcat = jnp.concatenate(
            [alo_ref[...] + tlo_ref[...], ahi_ref[...] + thi_ref[...]], axis=1)
        hact = jnp.maximum(hcat * dinv + b_ref[...], 0.0)
        t = _hp_dot(hact, w_ref[...])
        if split_out:
            out_refs[0][...] = t[:, :oh] * dinv
            out_refs[1][...] = t[:, oh:] * dinv
        else:
            out_refs[0][...] = t * dinv

    if split_out:
        out_specs = [pl.BlockSpec((_BN, oh), lambda i: (i, 0)),
                     pl.BlockSpec((_BN, oh), lambda i: (i, 0))]
        out_shape = [jax.ShapeDtypeStruct((n, oh), jnp.float32),
                     jax.ShapeDtypeStruct((n, oh), jnp.float32)]
    else:
        out_specs = [pl.BlockSpec((_BN, h_out), lambda i: (i, 0))]
        out_shape = [jax.ShapeDtypeStruct((n, h_out), jnp.float32)]

    return pl.pallas_call(
        body,
        grid=(n // _BN,),
        in_specs=[
            pl.BlockSpec((_BN, ih), lambda i: (i, 0)),
            pl.BlockSpec((_BN, ih), lambda i: (i, 0)),
            pl.BlockSpec((_BN, ih), lambda i: (i, 0)),
            pl.BlockSpec((_BN, ih), lambda i: (i, 0)),
            pl.BlockSpec((1, h_in), lambda i: (0, 0)),
            pl.BlockSpec((h_in, h_out), lambda i: (0, 0)),
            pl.BlockSpec((_BN, 16), lambda i: (i, 0)),
        ],
        out_specs=out_specs,
        out_shape=out_shape,
    )


@functools.lru_cache(None)
def _tc_last(n, h_out, mh):
    nblk = n // _BN

    def body(p0_ref, p1_ref, t_ref, b_ref, dinv_ref,
             bat_ref, a1_ref, c1_ref, a2_ref, c2_ref,
             h_ref, aout_ref, seg_ref, cnt_ref):
        i = pl.program_id(0)
        dinv = dinv_ref[...][:, 0:1]
        hcat = p0_ref[...] + p1_ref[...] + t_ref[...]
        hblk = hcat * dinv + b_ref[...]
        h_ref[...] = hblk

        bb = bat_ref[0, 0, :]
        onehot = (bb[:, None] ==
                  lax.broadcasted_iota(jnp.int32, (_BN, _G), 1)
                  ).astype(jnp.float32)
        seg_inc = lax.dot_general(onehot, hblk, (((0,), (0,)), ((), ())),
                                  precision=lax.Precision.HIGHEST,
                                  preferred_element_type=jnp.float32)
        cnt_inc = jnp.sum(onehot, axis=0)[None, :]

        @pl.when(i == 0)
        def _():
            seg_ref[...] = jnp.zeros_like(seg_ref)
            cnt_ref[...] = jnp.zeros_like(cnt_ref)

        seg_ref[...] += seg_inc
        cnt_ref[...] += cnt_inc

        @pl.when(i == nblk - 1)
        def _():
            cnt = jnp.maximum(cnt_ref[0, :], 1.0)
            gmean = seg_ref[...] / cnt[:, None]
            z = jnp.maximum(_hp_dot(gmean, a1_ref[...]) + c1_ref[...], 0.0)
            z2 = _hp_dot(z, a2_ref[...]) + c2_ref[...]
            aout_ref[...] = jax.nn.sigmoid(z2)

    return pl.pallas_call(
        body,
        grid=(nblk,),
        in_specs=[
            pl.BlockSpec((_BN, h_out), lambda i: (i, 0)),
            pl.BlockSpec((_BN, h_out), lambda i: (i, 0)),
            pl.BlockSpec((_BN, h_out), lambda i: (i, 0)),
            pl.BlockSpec((1, h_out), lambda i: (0, 0)),
            pl.BlockSpec((_BN, 16), lambda i: (i, 0)),
            pl.BlockSpec((1, 1, _BN), lambda i: (i, 0, 0)),
            pl.BlockSpec((h_out, mh), lambda i: (0, 0)),
            pl.BlockSpec((1, mh), lambda i: (0, 0)),
            pl.BlockSpec((mh, 1), lambda i: (0, 0)),
            pl.BlockSpec((1, 1), lambda i: (0, 0)),
        ],
        out_specs=[
            pl.BlockSpec((_BN, h_out), lambda i: (i, 0)),
            pl.BlockSpec((_G, 1), lambda i: (0, 0)),
        ],
        out_shape=[jax.ShapeDtypeStruct((n, h_out), jnp.float32),
                   jax.ShapeDtypeStruct((_G, 1), jnp.float32)],
        scratch_shapes=[pltpu.VMEM((_G, h_out), jnp.float32),
                        pltpu.VMEM((1, _G), jnp.float32)],
    )


def kernel(x, edge_index, batch, W1, b1, W2, b2, W3, b3, A1, c1, A2, c2):
    n, f_in = x.shape
    e = edge_index.shape[1]
    h = W1.shape[1]
    out = W3.shape[1]
    mh = A1.shape[1]

    src = edge_index[0]
    dst = edge_index[1]

    deg0, deg1 = _deg_kernel(n, e)(dst)

    t1lo, t1hi, dinv = _tc_first(n, f_in, h)(x, W1, deg0, deg1)
    a1lo, a1hi = _prop_kernel(n, e, h // 2)(src, dst, t1lo, t1hi)

    t2lo, t2hi = _tc_mid(n, h, h, True)(a1lo, a1hi, t1lo, t1hi,
                                        b1.reshape(1, h), W2, dinv)
    a2lo, a2hi = _prop_kernel(n, e, h // 2)(src, dst, t2lo, t2hi)

    (t3,) = _tc_mid(n, h, out, False)(a2lo, a2hi, t2lo, t2hi,
                                      b2.reshape(1, h), W3, dinv)
    a3p0, a3p1 = _prop_edge_split(n, e, out)(src, dst, t3)

    hfinal, a = _tc_last(n, out, mh)(
        a3p0, a3p1, t3, b3.reshape(1, out), dinv,
        batch.reshape(n // _BN, 1, _BN),
        A1, c1.reshape(1, mh), A2, c2.reshape(1, 1))
    return (hfinal, a)


# speedup vs baseline: 23.3070x; 1.0052x over previous
import functools

import jax
import jax.numpy as jnp
from jax import lax
from jax.experimental import pallas as pl
from jax.experimental.pallas import tpu as pltpu
from jax.experimental.pallas import tpu_sc as plsc

_NSUB = 16
_CH = 80
_ZR = 16
_G = 16
_BN = 1000


def _hp_dot(a, b):
    return jnp.dot(a, b, precision=lax.Precision.HIGHEST,
                   preferred_element_type=jnp.float32)


def _h_dot(a, b):
    return _hp_dot(a, b)


def _zero_acc(acc, zero_v, s, base, rpt, tail, w):
    @pl.loop(0, _ZR)
    def _(i):
        @pl.loop(0, w, step=16)
        def _(j):
            zero_v[i, pl.ds(j, 16)] = jnp.zeros((16,), jnp.float32)

    @pl.loop(0, rpt // _ZR)
    def _(k):
        pltpu.sync_copy(zero_v, acc.at[pl.ds(base + k * _ZR, _ZR)])

    if tail:
        @pl.when(s == _NSUB - 1)
        def _():
            @pl.loop(0, tail // _ZR)
            def _(k):
                pltpu.sync_copy(zero_v,
                                acc.at[pl.ds(rpt * _NSUB + k * _ZR, _ZR)])


def _flush_acc(acc, o_hbm, s, base, rpt, tail):
    pltpu.sync_copy(acc.at[pl.ds(base, rpt)], o_hbm.at[pl.ds(base, rpt)])
    if tail:
        @pl.when(s == _NSUB - 1)
        def _():
            pltpu.sync_copy(acc.at[pl.ds(rpt * _NSUB, tail)],
                            o_hbm.at[pl.ds(rpt * _NSUB, tail)])


@functools.lru_cache(None)
def _deg_kernel(n, e):
    rpt = (n // _NSUB) // 8 * 8
    tail = n - rpt * _NSUB
    assert rpt % _ZR == 0 and tail % _ZR == 0 and tail >= 0
    epw = e // (2 * _NSUB)
    nchunk = epw // _CH
    last = nchunk - 1
    mesh = plsc.VectorSubcoreMesh(core_axis_name="c", subcore_axis_name="s")

    @functools.partial(
        pl.kernel,
        out_type=[jax.ShapeDtypeStruct((n, 128), jnp.float32),
                  jax.ShapeDtypeStruct((n, 128), jnp.float32)],
        mesh=mesh,
        scratch_types=[
            pltpu.VMEM((4, _CH), jnp.int32),
            pltpu.VMEM((_CH, 128), jnp.float32),
            pltpu.VMEM((_ZR, 128), jnp.float32),
            pltpu.VMEM_SHARED((n, 128), jnp.float32),
        ] + [pltpu.SemaphoreType.DMA] * 8,
    )
    def deg(dst_hbm, deg0_hbm, deg1_hbm, dst_v, ones_v, zero_v, acc, *sems):
        c = lax.axis_index("c")
        s = lax.axis_index("s")
        ss, si = sems[0:4], sems[4:8]

        @pl.loop(0, _CH)
        def _(i):
            @pl.loop(0, 128, step=16)
            def _(j):
                ones_v[i, pl.ds(j, 16)] = jnp.ones((16,), jnp.float32)

        base = s * rpt
        _zero_acc(acc, zero_v, s, base, rpt, tail, 128)
        plsc.subcore_barrier()

        ebase = (c * _NSUB + s) * epw

        def i_start(j, b):
            pltpu.make_async_copy(dst_hbm.at[pl.ds(ebase + j * _CH, _CH)],
                                  dst_v.at[b], si[b]).start()

        def i_wait(b):
            pltpu.make_async_copy(dst_hbm.at[pl.ds(ebase, _CH)],
                                  dst_v.at[b], si[b]).wait()

        def s_start(b):
            pltpu.async_copy(ones_v, acc.at[dst_v.at[b]], ss[b], add=True)

        def s_wait(b):
            pltpu.make_async_copy(ones_v, acc.at[dst_v.at[b]], ss[b]).wait()

        def make_slot(k):
            b, b2 = k % 4, (k + 2) % 4

            def run_slot(j, jp):
                i_wait(b)
                s_start(b)
                s_wait(b2)
                i_start(jp, b2)
            return run_slot

        slots = [make_slot(k) for k in range(4)]

        pltpu.sync_copy(dst_hbm.at[pl.ds(ebase, _CH)], dst_v.at[0])
        pltpu.sync_copy(dst_hbm.at[pl.ds(ebase + _CH, _CH)], dst_v.at[1])
        i_start(2, 2)
        i_start(3, 3)
        s_start(0)
        s_start(1)

        n_slots = nchunk - 2
        n_iter = n_slots // 4

        @pl.loop(0, n_iter)
        def _(jj):
            j0 = 2 + 4 * jj
            for t in range(4):
                slots[(2 + t) % 4](j0 + t, jnp.minimum(j0 + t + 2, last))

        for t in range(n_slots - 4 * n_iter):
            j = 2 + 4 * n_iter + t
            slots[j % 4](j, min(j + 2, last))

        i_wait((last + 1) % 4)
        i_wait((last + 2) % 4)
        s_wait((last - 1) % 4)
        s_wait(last % 4)

        plsc.subcore_barrier()

        @pl.when(c == 0)
        def _():
            _flush_acc(acc, deg0_hbm, s, base, rpt, tail)

        @pl.when(c == 1)
        def _():
            _flush_acc(acc, deg1_hbm, s, base, rpt, tail)

    return deg


def _edge_pipeline(src_hbm, dst_hbm, t_hbm, acc, ebase, nchunk,
                   src_v, dst_v, rows, sg, ss, si):
    last = nchunk - 1
    assert nchunk >= 4

    def i_start(j, b, sem):
        pltpu.make_async_copy(src_hbm.at[pl.ds(ebase + j * _CH, _CH)],
                              src_v.at[b], sem).start()
        pltpu.make_async_copy(dst_hbm.at[pl.ds(ebase + j * _CH, _CH)],
                              dst_v.at[b], sem).start()

    def i_sync(j, b):
        pltpu.sync_copy(src_hbm.at[pl.ds(ebase + j * _CH, _CH)], src_v.at[b])
        pltpu.sync_copy(dst_hbm.at[pl.ds(ebase + j * _CH, _CH)], dst_v.at[b])

    def i_wait(b):
        pltpu.make_async_copy(src_hbm.at[pl.ds(ebase, _CH)],
                              src_v.at[b], si[b]).wait()
        pltpu.make_async_copy(dst_hbm.at[pl.ds(ebase, _CH)],
                              dst_v.at[b], si[b]).wait()

    def g_start(b):
        pltpu.make_async_copy(t_hbm.at[src_v.at[b]], rows[b], sg[b]).start()

    def g_wait(b):
        pltpu.make_async_copy(t_hbm.at[src_v.at[b]], rows[b], sg[b]).wait()

    def s_start(b):
        pltpu.async_copy(rows[b], acc.at[dst_v.at[b]], ss[b], add=True)

    def s_wait(b):
        pltpu.make_async_copy(rows[b], acc.at[dst_v.at[b]], ss[b]).wait()

    def make_slot(k):
        b, b1, b2 = k % 4, (k - 1) % 4, (k + 2) % 4

        def run_slot(j, jp):
            i_wait(b)
            g_start(b)
            g_wait(b1)
            s_start(b1)
            s_wait(b2)
            i_start(jp, b2, si[b2])
        return run_slot

    slots = [make_slot(k) for k in range(4)]

    i_sync(0, 0)
    i_sync(1, 1)
    i_start(2, 2, si[2])
    i_start(3, 3, si[3])
    g_start(0)
    g_start(1)
    g_wait(0)
    s_start(0)

    n_slots = nchunk - 2
    n_iter = n_slots // 4

    @pl.loop(0, n_iter)
    def _(jj):
        j0 = 2 + 4 * jj
        for t in range(4):
            slots[(2 + t) % 4](j0 + t, jnp.minimum(j0 + t + 2, last))

    for t in range(n_slots - 4 * n_iter):
        j = 2 + 4 * n_iter + t
        slots[j % 4](j, min(j + 2, last))

    bL = last % 4
    g_wait(bL)
    s_start(bL)
    i_wait((last + 1) % 4)
    i_wait((last + 2) % 4)
    s_wait((last - 1) % 4)
    s_wait(bL)


@functools.lru_cache(None)
def _prop_kernel(n, e, w):
    rpt = (n // _NSUB) // 8 * 8
    tail = n - rpt * _NSUB
    assert rpt % _ZR == 0 and tail % _ZR == 0 and tail >= 0
    epw = e // _NSUB
    nchunk = epw // _CH
    mesh = plsc.VectorSubcoreMesh(core_axis_name="c", subcore_axis_name="s")

    @functools.partial(
        pl.kernel,
        out_type=[jax.ShapeDtypeStruct((n, w), jnp.float32),
                  jax.ShapeDtypeStruct((n, w), jnp.float32)],
        mesh=mesh,
        scratch_types=[
            pltpu.VMEM((4, _CH), jnp.int32),
            pltpu.VMEM((4, _CH), jnp.int32),
            pltpu.VMEM((_CH, w), jnp.float32),
            pltpu.VMEM((_CH, w), jnp.float32),
            pltpu.VMEM((_CH, w), jnp.float32),
            pltpu.VMEM((_CH, w), jnp.float32),
            pltpu.VMEM((_ZR, w), jnp.float32),
            pltpu.VMEM_SHARED((n, w), jnp.float32),
        ] + [pltpu.SemaphoreType.DMA] * 12,
    )
    def prop(src_hbm, dst_hbm, tlo_hbm, thi_hbm, olo_hbm, ohi_hbm,
             src_v, dst_v, r0, r1, r2, r3, zero_v, acc, *sems):
        c = lax.axis_index("c")
        s = lax.axis_index("s")
        rows, sg, ss, si = [r0, r1, r2, r3], sems[0:4], sems[4:8], sems[8:12]

        base = s * rpt
        _zero_acc(acc, zero_v, s, base, rpt, tail, w)
        plsc.subcore_barrier()

        ebase = s * epw

        def run(t_hbm, o_hbm):
            _edge_pipeline(src_hbm, dst_hbm, t_hbm, acc, ebase, nchunk,
                           src_v, dst_v, rows, sg, ss, si)
            plsc.subcore_barrier()
            _flush_acc(acc, o_hbm, s, base, rpt, tail)

        @pl.when(c == 0)
        def _():
            run(tlo_hbm, olo_hbm)

        @pl.when(c == 1)
        def _():
            run(thi_hbm, ohi_hbm)

    return prop


@functools.lru_cache(None)
def _prop_edge_split(n, e, w):
    rpt = (n // _NSUB) // 8 * 8
    tail = n - rpt * _NSUB
    assert rpt % _ZR == 0 and tail % _ZR == 0 and tail >= 0
    epw = e // (2 * _NSUB)
    nchunk = epw // _CH
    mesh = plsc.VectorSubcoreMesh(core_axis_name="c", subcore_axis_name="s")

    @functools.partial(
        pl.kernel,
        out_type=[jax.ShapeDtypeStruct((n, w), jnp.float32),
                  jax.ShapeDtypeStruct((n, w), jnp.float32)],
        mesh=mesh,
        scratch_types=[
            pltpu.VMEM((4, _CH), jnp.int32),
            pltpu.VMEM((4, _CH), jnp.int32),
            pltpu.VMEM((_CH, w), jnp.float32),
            pltpu.VMEM((_CH, w), jnp.float32),
            pltpu.VMEM((_CH, w), jnp.float32),
            pltpu.VMEM((_CH, w), jnp.float32),
            pltpu.VMEM((_ZR, w), jnp.float32),
            pltpu.VMEM_SHARED((n, w), jnp.float32),
        ] + [pltpu.SemaphoreType.DMA] * 12,
    )
    def prop(src_hbm, dst_hbm, t_hbm, o0_hbm, o1_hbm,
             src_v, dst_v, r0, r1, r2, r3, zero_v, acc, *sems):
        c = lax.axis_index("c")
        s = lax.axis_index("s")
        rows, sg, ss, si = [r0, r1, r2, r3], sems[0:4], sems[4:8], sems[8:12]

        base = s * rpt
        _zero_acc(acc, zero_v, s, base, rpt, tail, w)
        plsc.subcore_barrier()

        ebase = (c * _NSUB + s) * epw

        _edge_pipeline(src_hbm, dst_hbm, t_hbm, acc, ebase, nchunk,
                       src_v, dst_v, rows, sg, ss, si)
        plsc.subcore_barrier()

        @pl.when(c == 0)
        def _():
            _flush_acc(acc, o0_hbm, s, base, rpt, tail)

        @pl.when(c == 1)
        def _():
            _flush_acc(acc, o1_hbm, s, base, rpt, tail)

    return prop


@functools.lru_cache(None)
def _tc_mm_first(n, f_in, h):
    def body(x_ref, w_ref, t_ref):
        t_ref[...] = _h_dot(x_ref[...], w_ref[...])

    return pl.pallas_call(
        body,
        grid=(n // _BN,),
        in_specs=[
            pl.BlockSpec((_BN, f_in), lambda i: (i, 0)),
            pl.BlockSpec((f_in, h), lambda i: (0, 0)),
        ],
        out_specs=[pl.BlockSpec((_BN, h), lambda i: (i, 0))],
        out_shape=[jax.ShapeDtypeStruct((n, h), jnp.float32)],
    )


@functools.lru_cache(None)
def _tc_first(n, h):
    hh = h // 2

    def body(t_ref, d0_ref, d1_ref, lo_ref, hi_ref, dinv_ref):
        deg = d0_ref[...][:, 0:1] + d1_ref[...][:, 0:1] + 1.0
        dinv = lax.rsqrt(deg)
        t = t_ref[...]
        lo_ref[...] = t[:, :hh] * dinv
        hi_ref[...] = t[:, hh:] * dinv
        dinv_ref[...] = jnp.broadcast_to(dinv, (_BN, 16))

    return pl.pallas_call(
        body,
        grid=(n // _BN,),
        in_specs=[
            pl.BlockSpec((_BN, h), lambda i: (i, 0)),
            pl.BlockSpec((_BN, 128), lambda i: (i, 0)),
            pl.BlockSpec((_BN, 128), lambda i: (i, 0)),
        ],
        out_specs=[
            pl.BlockSpec((_BN, hh), lambda i: (i, 0)),
            pl.BlockSpec((_BN, hh), lambda i: (i, 0)),
            pl.BlockSpec((_BN, 16), lambda i: (i, 0)),
        ],
        out_shape=[jax.ShapeDtypeStruct((n, hh), jnp.float32),
                   jax.ShapeDtypeStruct((n, hh), jnp.float32),
                   jax.ShapeDtypeStruct((n, 16), jnp.float32)],
    )


@functools.lru_cache(None)
def _tc_mid(n, h_in, h_out, split_out):
    ih = h_in // 2
    oh = h_out // 2

    def body(alo_ref, ahi_ref, tlo_ref, thi_ref, b_ref, w_ref,
             dinv_ref, *out_refs):
        dinv = dinv_ref[...][:, 0:1]
        hcat = jnp.concatenate(
            [alo_ref[...] + tlo_ref[...], ahi_ref[...] + thi_ref[...]], axis=1)
        hact = jnp.maximum(hcat * dinv + b_ref[...], 0.0)
        t = _h_dot(hact, w_ref[...])
        if split_out:
            out_refs[0][...] = t[:, :oh] * dinv
            out_refs[1][...] = t[:, oh:] * dinv
        else:
            out_refs[0][...] = t * dinv

    if split_out:
        out_specs = [pl.BlockSpec((_BN, oh), lambda i: (i, 0)),
                     pl.BlockSpec((_BN, oh), lambda i: (i, 0))]
        out_shape = [jax.ShapeDtypeStruct((n, oh), jnp.float32),
                     jax.ShapeDtypeStruct((n, oh), jnp.float32)]
    else:
        out_specs = [pl.BlockSpec((_BN, h_out), lambda i: (i, 0))]
        out_shape = [jax.ShapeDtypeStruct((n, h_out), jnp.float32)]

    return pl.pallas_call(
        body,
        grid=(n // _BN,),
        in_specs=[
            pl.BlockSpec((_BN, ih), lambda i: (i, 0)),
            pl.BlockSpec((_BN, ih), lambda i: (i, 0)),
            pl.BlockSpec((_BN, ih), lambda i: (i, 0)),
            pl.BlockSpec((_BN, ih), lambda i: (i, 0)),
            pl.BlockSpec((1, h_in), lambda i: (0, 0)),
            pl.BlockSpec((h_in, h_out), lambda i: (0, 0)),
            pl.BlockSpec((_BN, 16), lambda i: (i, 0)),
        ],
        out_specs=out_specs,
        out_shape=out_shape,
    )


@functools.lru_cache(None)
def _tc_last(n, h_out, mh):
    nblk = n // _BN

    def body(p0_ref, p1_ref, t_ref, b_ref, dinv_ref,
             bat_ref, a1_ref, c1_ref, a2_ref, c2_ref,
             h_ref, aout_ref, seg_ref, cnt_ref):
        i = pl.program_id(0)
        dinv = dinv_ref[...][:, 0:1]
        hcat = p0_ref[...] + p1_ref[...] + t_ref[...]
        hblk = hcat * dinv + b_ref[...]
        h_ref[...] = hblk

        bb = bat_ref[0, 0, :]
        onehot = (bb[:, None] ==
                  lax.broadcasted_iota(jnp.int32, (_BN, _G), 1)
                  ).astype(jnp.float32)
        seg_inc = lax.dot_general(onehot, hblk, (((0,), (0,)), ((), ())),
                                  precision=lax.Precision.HIGHEST,
                                  preferred_element_type=jnp.float32)
        cnt_inc = jnp.sum(onehot, axis=0)[None, :]

        @pl.when(i == 0)
        def _():
            seg_ref[...] = jnp.zeros_like(seg_ref)
            cnt_ref[...] = jnp.zeros_like(cnt_ref)

        seg_ref[...] += seg_inc
        cnt_ref[...] += cnt_inc

        @pl.when(i == nblk - 1)
        def _():
            cnt = jnp.maximum(cnt_ref[0, :], 1.0)
            gmean = seg_ref[...] / cnt[:, None]
            z = jnp.maximum(_hp_dot(gmean, a1_ref[...]) + c1_ref[...], 0.0)
            z2 = _hp_dot(z, a2_ref[...]) + c2_ref[...]
            aout_ref[...] = jax.nn.sigmoid(z2)

    return pl.pallas_call(
        body,
        grid=(nblk,),
        in_specs=[
            pl.BlockSpec((_BN, h_out), lambda i: (i, 0)),
            pl.BlockSpec((_BN, h_out), lambda i: (i, 0)),
            pl.BlockSpec((_BN, h_out), lambda i: (i, 0)),
            pl.BlockSpec((1, h_out), lambda i: (0, 0)),
            pl.BlockSpec((_BN, 16), lambda i: (i, 0)),
            pl.BlockSpec((1, 1, _BN), lambda i: (i, 0, 0)),
            pl.BlockSpec((h_out, mh), lambda i: (0, 0)),
            pl.BlockSpec((1, mh), lambda i: (0, 0)),
            pl.BlockSpec((mh, 1), lambda i: (0, 0)),
            pl.BlockSpec((1, 1), lambda i: (0, 0)),
        ],
        out_specs=[
            pl.BlockSpec((_BN, h_out), lambda i: (i, 0)),
            pl.BlockSpec((_G, 1), lambda i: (0, 0)),
        ],
        out_shape=[jax.ShapeDtypeStruct((n, h_out), jnp.float32),
                   jax.ShapeDtypeStruct((_G, 1), jnp.float32)],
        scratch_shapes=[pltpu.VMEM((_G, h_out), jnp.float32),
                        pltpu.VMEM((1, _G), jnp.float32)],
    )


def kernel(x, edge_index, batch, W1, b1, W2, b2, W3, b3, A1, c1, A2, c2):
    n, f_in = x.shape
    e = edge_index.shape[1]
    h = W1.shape[1]
    out = W3.shape[1]
    mh = A1.shape[1]

    src = edge_index[0]
    dst = edge_index[1]

    (t1raw,) = _tc_mm_first(n, f_in, h)(x, W1)
    deg0, deg1 = _deg_kernel(n, e)(dst)

    t1lo, t1hi, dinv = _tc_first(n, h)(t1raw, deg0, deg1)
    a1lo, a1hi = _prop_kernel(n, e, h // 2)(src, dst, t1lo, t1hi)

    t2lo, t2hi = _tc_mid(n, h, h, True)(a1lo, a1hi, t1lo, t1hi,
                                        b1.reshape(1, h), W2, dinv)
    a2lo, a2hi = _prop_kernel(n, e, h // 2)(src, dst, t2lo, t2hi)

    (t3,) = _tc_mid(n, h, out, False)(a2lo, a2hi, t2lo, t2hi,
                                      b2.reshape(1, h), W3, dinv)
    a3p0, a3p1 = _prop_edge_split(n, e, out)(src, dst, t3)

    hfinal, a = _tc_last(n, out, mh)(
        a3p0, a3p1, t3, b3.reshape(1, out), dinv,
        batch.reshape(n // _BN, 1, _BN),
        A1, c1.reshape(1, mh), A2, c2.reshape(1, 1))
    return (hfinal, a)


# speedup vs baseline: 23.8208x; 1.0220x over previous
import functools

import jax
import jax.numpy as jnp
from jax import lax
from jax.experimental import pallas as pl
from jax.experimental.pallas import tpu as pltpu
from jax.experimental.pallas import tpu_sc as plsc

_NSUB = 16
_CH = 96
_G = 16
_BN = 1000


def _hp_dot(a, b):
    return jnp.dot(a, b, precision=lax.Precision.HIGHEST,
                   preferred_element_type=jnp.float32)


def _h_dot(a, b):
    return _hp_dot(a, b)


def _fill(buf, rows, w, val):
    @pl.loop(0, rows)
    def _(i):
        @pl.loop(0, w, step=16)
        def _(j):
            buf[i, pl.ds(j, 16)] = jnp.full((16,), val, jnp.float32)


def _zero_acc(acc, zbuf, s, base, rpt, tail, w):
    _fill(zbuf, _CH, w, 0.0)
    nz = rpt // _CH
    rem = rpt - nz * _CH

    @pl.loop(0, nz)
    def _(k):
        pltpu.sync_copy(zbuf, acc.at[pl.ds(base + k * _CH, _CH)])

    if rem:
        pltpu.sync_copy(zbuf.at[pl.ds(0, rem)],
                        acc.at[pl.ds(base + nz * _CH, rem)])
    if tail:
        @pl.when(s == _NSUB - 1)
        def _():
            pltpu.sync_copy(zbuf.at[pl.ds(0, tail)],
                            acc.at[pl.ds(rpt * _NSUB, tail)])


def _flush_acc(acc, o_hbm, s, base, rpt, tail):
    pltpu.sync_copy(acc.at[pl.ds(base, rpt)], o_hbm.at[pl.ds(base, rpt)])
    if tail:
        @pl.when(s == _NSUB - 1)
        def _():
            pltpu.sync_copy(acc.at[pl.ds(rpt * _NSUB, tail)],
                            o_hbm.at[pl.ds(rpt * _NSUB, tail)])


@functools.lru_cache(None)
def _deg_kernel(n, e):
    rpt = (n // _NSUB) // 8 * 8
    tail = n - rpt * _NSUB
    assert tail % 8 == 0 and 0 <= tail <= _CH and rpt >= _CH
    epw = e // (2 * _NSUB)
    nchunk = epw // _CH
    tr = epw - nchunk * _CH
    assert tr % 8 == 0
    last = nchunk - 1
    mesh = plsc.VectorSubcoreMesh(core_axis_name="c", subcore_axis_name="s")

    @functools.partial(
        pl.kernel,
        out_type=[jax.ShapeDtypeStruct((n, 128), jnp.float32),
                  jax.ShapeDtypeStruct((n, 128), jnp.float32)],
        mesh=mesh,
        scratch_types=[
            pltpu.VMEM((4, _CH), jnp.int32),
            pltpu.VMEM((1, max(tr, 8)), jnp.int32),
            pltpu.VMEM((_CH, 128), jnp.float32),
            pltpu.VMEM_SHARED((n, 128), jnp.float32),
        ] + [pltpu.SemaphoreType.DMA] * 8,
    )
    def deg(dst_hbm, deg0_hbm, deg1_hbm, dst_v, tdst_v, ones_v, acc, *sems):
        c = lax.axis_index("c")
        s = lax.axis_index("s")
        ss, si = sems[0:4], sems[4:8]

        base = s * rpt
        _zero_acc(acc, ones_v, s, base, rpt, tail, 128)
        _fill(ones_v, _CH, 128, 1.0)
        plsc.subcore_barrier()

        ebase = (c * _NSUB + s) * epw

        def i_start(j, b):
            pltpu.make_async_copy(dst_hbm.at[pl.ds(ebase + j * _CH, _CH)],
                                  dst_v.at[b], si[b]).start()

        def i_wait(b):
            pltpu.make_async_copy(dst_hbm.at[pl.ds(ebase, _CH)],
                                  dst_v.at[b], si[b]).wait()

        def s_start(b):
            pltpu.async_copy(ones_v, acc.at[dst_v.at[b]], ss[b], add=True)

        def s_wait(b):
            pltpu.make_async_copy(ones_v, acc.at[dst_v.at[b]], ss[b]).wait()

        def make_slot(k):
            b, b2 = k % 4, (k + 2) % 4

            def run_slot(j, jp):
                i_wait(b)
                s_start(b)
                s_wait(b2)
                i_start(jp, b2)
            return run_slot

        slots = [make_slot(k) for k in range(4)]

        pltpu.sync_copy(dst_hbm.at[pl.ds(ebase, _CH)], dst_v.at[0])
        pltpu.sync_copy(dst_hbm.at[pl.ds(ebase + _CH, _CH)], dst_v.at[1])
        i_start(2, 2)
        i_start(3, 3)
        s_start(0)
        s_start(1)

        n_slots = nchunk - 2
        n_iter = n_slots // 4

        @pl.loop(0, n_iter)
        def _(jj):
            j0 = 2 + 4 * jj
            for t in range(4):
                slots[(2 + t) % 4](j0 + t, jnp.minimum(j0 + t + 2, last))

        for t in range(n_slots - 4 * n_iter):
            j = 2 + 4 * n_iter + t
            slots[j % 4](j, min(j + 2, last))

        i_wait((last + 1) % 4)
        i_wait((last + 2) % 4)
        s_wait((last - 1) % 4)
        s_wait(last % 4)

        if tr:
            off = ebase + nchunk * _CH
            pltpu.sync_copy(dst_hbm.at[pl.ds(off, tr)], tdst_v.at[0])
            pltpu.sync_copy(ones_v.at[pl.ds(0, tr)],
                            acc.at[tdst_v.at[0]], add=True)

        plsc.subcore_barrier()

        @pl.when(c == 0)
        def _():
            _flush_acc(acc, deg0_hbm, s, base, rpt, tail)

        @pl.when(c == 1)
        def _():
            _flush_acc(acc, deg1_hbm, s, base, rpt, tail)

    return deg


def _edge_pipeline(src_hbm, dst_hbm, t_hbm, acc, ebase, nchunk, tr,
                   src_v, dst_v, tsrc_v, tdst_v, rows, sg, ss, si):
    last = nchunk - 1
    assert nchunk >= 4

    def i_start(j, b, sem):
        pltpu.make_async_copy(src_hbm.at[pl.ds(ebase + j * _CH, _CH)],
                              src_v.at[b], sem).start()
        pltpu.make_async_copy(dst_hbm.at[pl.ds(ebase + j * _CH, _CH)],
                              dst_v.at[b], sem).start()

    def i_sync(j, b):
        pltpu.sync_copy(src_hbm.at[pl.ds(ebase + j * _CH, _CH)], src_v.at[b])
        pltpu.sync_copy(dst_hbm.at[pl.ds(ebase + j * _CH, _CH)], dst_v.at[b])

    def i_wait(b):
        pltpu.make_async_copy(src_hbm.at[pl.ds(ebase, _CH)],
                              src_v.at[b], si[b]).wait()
        pltpu.make_async_copy(dst_hbm.at[pl.ds(ebase, _CH)],
                              dst_v.at[b], si[b]).wait()

    def g_start(b):
        pltpu.make_async_copy(t_hbm.at[src_v.at[b]], rows[b], sg[b]).start()

    def g_wait(b):
        pltpu.make_async_copy(t_hbm.at[src_v.at[b]], rows[b], sg[b]).wait()

    def s_start(b):
        pltpu.async_copy(rows[b], acc.at[dst_v.at[b]], ss[b], add=True)

    def s_wait(b):
        pltpu.make_async_copy(rows[b], acc.at[dst_v.at[b]], ss[b]).wait()

    def make_slot(k):
        b, b1, b2 = k % 4, (k - 1) % 4, (k + 2) % 4

        def run_slot(j, jp):
            i_wait(b)
            g_start(b)
            g_wait(b1)
            s_start(b1)
            s_wait(b2)
            i_start(jp, b2, si[b2])
        return run_slot

    slots = [make_slot(k) for k in range(4)]

    i_sync(0, 0)
    i_sync(1, 1)
    i_start(2, 2, si[2])
    i_start(3, 3, si[3])
    g_start(0)
    g_start(1)
    g_wait(0)
    s_start(0)

    n_slots = nchunk - 2
    n_iter = n_slots // 4

    @pl.loop(0, n_iter)
    def _(jj):
        j0 = 2 + 4 * jj
        for t in range(4):
            slots[(2 + t) % 4](j0 + t, jnp.minimum(j0 + t + 2, last))

    for t in range(n_slots - 4 * n_iter):
        j = 2 + 4 * n_iter + t
        slots[j % 4](j, min(j + 2, last))

    bL = last % 4
    g_wait(bL)
    s_start(bL)
    i_wait((last + 1) % 4)
    i_wait((last + 2) % 4)
    s_wait((last - 1) % 4)
    s_wait(bL)

    if tr:
        off = ebase + nchunk * _CH
        pltpu.sync_copy(src_hbm.at[pl.ds(off, tr)], tsrc_v)
        pltpu.sync_copy(dst_hbm.at[pl.ds(off, tr)], tdst_v.at[0])
        pltpu.sync_copy(t_hbm.at[tsrc_v], rows[0].at[pl.ds(0, tr)])
        pltpu.sync_copy(rows[0].at[pl.ds(0, tr)],
                        acc.at[tdst_v.at[0]], add=True)


@functools.lru_cache(None)
def _prop_kernel(n, e, w):
    rpt = (n // _NSUB) // 8 * 8
    tail = n - rpt * _NSUB
    assert tail % 8 == 0 and 0 <= tail <= _CH and rpt >= _CH
    epw = e // _NSUB
    nchunk = epw // _CH
    tr = epw - nchunk * _CH
    assert tr % 8 == 0
    mesh = plsc.VectorSubcoreMesh(core_axis_name="c", subcore_axis_name="s")

    @functools.partial(
        pl.kernel,
        out_type=[jax.ShapeDtypeStruct((n, w), jnp.float32),
                  jax.ShapeDtypeStruct((n, w), jnp.float32)],
        mesh=mesh,
        scratch_types=[
            pltpu.VMEM((4, _CH), jnp.int32),
            pltpu.VMEM((4, _CH), jnp.int32),
            pltpu.VMEM((max(tr, 8),), jnp.int32),
            pltpu.VMEM((1, max(tr, 8)), jnp.int32),
            pltpu.VMEM((_CH, w), jnp.float32),
            pltpu.VMEM((_CH, w), jnp.float32),
            pltpu.VMEM((_CH, w), jnp.float32),
            pltpu.VMEM((_CH, w), jnp.float32),
            pltpu.VMEM_SHARED((n, w), jnp.float32),
        ] + [pltpu.SemaphoreType.DMA] * 12,
    )
    def prop(src_hbm, dst_hbm, tlo_hbm, thi_hbm, olo_hbm, ohi_hbm,
             src_v, dst_v, tsrc_v, tdst_v, r0, r1, r2, r3, acc, *sems):
        c = lax.axis_index("c")
        s = lax.axis_index("s")
        rows, sg, ss, si = [r0, r1, r2, r3], sems[0:4], sems[4:8], sems[8:12]

        base = s * rpt
        _zero_acc(acc, r0, s, base, rpt, tail, w)
        plsc.subcore_barrier()

        ebase = s * epw

        def run(t_hbm, o_hbm):
            _edge_pipeline(src_hbm, dst_hbm, t_hbm, acc, ebase, nchunk, tr,
                           src_v, dst_v, tsrc_v, tdst_v, rows, sg, ss, si)
            plsc.subcore_barrier()
            _flush_acc(acc, o_hbm, s, base, rpt, tail)

        @pl.when(c == 0)
        def _():
            run(tlo_hbm, olo_hbm)

        @pl.when(c == 1)
        def _():
            run(thi_hbm, ohi_hbm)

    return prop


@functools.lru_cache(None)
def _prop_edge_split(n, e, w):
    rpt = (n // _NSUB) // 8 * 8
    tail = n - rpt * _NSUB
    assert tail % 8 == 0 and 0 <= tail <= _CH and rpt >= _CH
    epw = e // (2 * _NSUB)
    nchunk = epw // _CH
    tr = epw - nchunk * _CH
    assert tr % 8 == 0
    mesh = plsc.VectorSubcoreMesh(core_axis_name="c", subcore_axis_name="s")

    @functools.partial(
        pl.kernel,
        out_type=[jax.ShapeDtypeStruct((n, w), jnp.float32),
                  jax.ShapeDtypeStruct((n, w), jnp.float32)],
        mesh=mesh,
        scratch_types=[
            pltpu.VMEM((4, _CH), jnp.int32),
            pltpu.VMEM((4, _CH), jnp.int32),
            pltpu.VMEM((max(tr, 8),), jnp.int32),
            pltpu.VMEM((1, max(tr, 8)), jnp.int32),
            pltpu.VMEM((_CH, w), jnp.float32),
            pltpu.VMEM((_CH, w), jnp.float32),
            pltpu.VMEM((_CH, w), jnp.float32),
            pltpu.VMEM((_CH, w), jnp.float32),
            pltpu.VMEM_SHARED((n, w), jnp.float32),
        ] + [pltpu.SemaphoreType.DMA] * 12,
    )
    def prop(src_hbm, dst_hbm, t_hbm, o0_hbm, o1_hbm,
             src_v, dst_v, tsrc_v, tdst_v, r0, r1, r2, r3, acc, *sems):
        c = lax.axis_index("c")
        s = lax.axis_index("s")
        rows, sg, ss, si = [r0, r1, r2, r3], sems[0:4], sems[4:8], sems[8:12]

        base = s * rpt
        _zero_acc(acc, r0, s, base, rpt, tail, w)
        plsc.subcore_barrier()

        ebase = (c * _NSUB + s) * epw

        _edge_pipeline(src_hbm, dst_hbm, t_hbm, acc, ebase, nchunk, tr,
                       src_v, dst_v, tsrc_v, tdst_v, rows, sg, ss, si)
        plsc.subcore_barrier()

        @pl.when(c == 0)
        def _():
            _flush_acc(acc, o0_hbm, s, base, rpt, tail)

        @pl.when(c == 1)
        def _():
            _flush_acc(acc, o1_hbm, s, base, rpt, tail)

    return prop


@functools.lru_cache(None)
def _tc_mm_first(n, f_in, h):
    def body(x_ref, w_ref, t_ref):
        t_ref[...] = _h_dot(x_ref[...], w_ref[...])

    return pl.pallas_call(
        body,
        grid=(n // _BN,),
        in_specs=[
            pl.BlockSpec((_BN, f_in), lambda i: (i, 0)),
            pl.BlockSpec((f_in, h), lambda i: (0, 0)),
        ],
        out_specs=[pl.BlockSpec((_BN, h), lambda i: (i, 0))],
        out_shape=[jax.ShapeDtypeStruct((n, h), jnp.float32)],
    )


@functools.lru_cache(None)
def _tc_first(n, h):
    hh = h // 2

    def body(t_ref, d0_ref, d1_ref, lo_ref, hi_ref, dinv_ref):
        deg = d0_ref[...][:, 0:1] + d1_ref[...][:, 0:1] + 1.0
        dinv = lax.rsqrt(deg)
        t = t_ref[...]
        lo_ref[...] = t[:, :hh] * dinv
        hi_ref[...] = t[:, hh:] * dinv
        dinv_ref[...] = jnp.broadcast_to(dinv, (_BN, 16))

    return pl.pallas_call(
        body,
        grid=(n // _BN,),
        in_specs=[
            pl.BlockSpec((_BN, h), lambda i: (i, 0)),
            pl.BlockSpec((_BN, 128), lambda i: (i, 0)),
            pl.BlockSpec((_BN, 128), lambda i: (i, 0)),
        ],
        out_specs=[
            pl.BlockSpec((_BN, hh), lambda i: (i, 0)),
            pl.BlockSpec((_BN, hh), lambda i: (i, 0)),
            pl.BlockSpec((_BN, 16), lambda i: (i, 0)),
        ],
        out_shape=[jax.ShapeDtypeStruct((n, hh), jnp.float32),
                   jax.ShapeDtypeStruct((n, hh), jnp.float32),
                   jax.ShapeDtypeStruct((n, 16), jnp.float32)],
    )


@functools.lru_cache(None)
def _tc_mid(n, h_in, h_out, split_out):
    ih = h_in // 2
    oh = h_out // 2

    def body(alo_ref, ahi_ref, tlo_ref, thi_ref, b_ref, w_ref,
             dinv_ref, *out_refs):
        dinv = dinv_ref[...][:, 0:1]
        hcat = jnp.concatenate(
            [alo_ref[...] + tlo_ref[...], ahi_ref[...] + thi_ref[...]], axis=1)
        hact = jnp.maximum(hcat * dinv + b_ref[...], 0.0)
        t = _h_dot(hact, w_ref[...])
        if split_out:
            out_refs[0][...] = t[:, :oh] * dinv
            out_refs[1][...] = t[:, oh:] * dinv
        else:
            out_refs[0][...] = t * dinv

    if split_out:
        out_specs = [pl.BlockSpec((_BN, oh), lambda i: (i, 0)),
                     pl.BlockSpec((_BN, oh), lambda i: (i, 0))]
        out_shape = [jax.ShapeDtypeStruct((n, oh), jnp.float32),
                     jax.ShapeDtypeStruct((n, oh), jnp.float32)]
    else:
        out_specs = [pl.BlockSpec((_BN, h_out), lambda i: (i, 0))]
        out_shape = [jax.ShapeDtypeStruct((n, h_out), jnp.float32)]

    return pl.pallas_call(
        body,
        grid=(n // _BN,),
        in_specs=[
            pl.BlockSpec((_BN, ih), lambda i: (i, 0)),
            pl.BlockSpec((_BN, ih), lambda i: (i, 0)),
            pl.BlockSpec((_BN, ih), lambda i: (i, 0)),
            pl.BlockSpec((_BN, ih), lambda i: (i, 0)),
            pl.BlockSpec((1, h_in), lambda i: (0, 0)),
            pl.BlockSpec((h_in, h_out), lambda i: (0, 0)),
            pl.BlockSpec((_BN, 16), lambda i: (i, 0)),
        ],
        out_specs=out_specs,
        out_shape=out_shape,
    )


@functools.lru_cache(None)
def _tc_last(n, h_out, mh):
    nblk = n // _BN

    def body(p0_ref, p1_ref, t_ref, b_ref, dinv_ref,
             bat_ref, a1_ref, c1_ref, a2_ref, c2_ref,
             h_ref, aout_ref, seg_ref, cnt_ref):
        i = pl.program_id(0)
        dinv = dinv_ref[...][:, 0:1]
        hcat = p0_ref[...] + p1_ref[...] + t_ref[...]
        hblk = hcat * dinv + b_ref[...]
        h_ref[...] = hblk

        bb = bat_ref[0, 0, :]
        onehot = (bb[:, None] ==
                  lax.broadcasted_iota(jnp.int32, (_BN, _G), 1)
                  ).astype(jnp.float32)
        seg_inc = lax.dot_general(onehot, hblk, (((0,), (0,)), ((), ())),
                                  precision=lax.Precision.HIGHEST,
                                  preferred_element_type=jnp.float32)
        cnt_inc = jnp.sum(onehot, axis=0)[None, :]

        @pl.when(i == 0)
        def _():
            seg_ref[...] = jnp.zeros_like(seg_ref)
            cnt_ref[...] = jnp.zeros_like(cnt_ref)

        seg_ref[...] += seg_inc
        cnt_ref[...] += cnt_inc

        @pl.when(i == nblk - 1)
        def _():
            cnt = jnp.maximum(cnt_ref[0, :], 1.0)
            gmean = seg_ref[...] / cnt[:, None]
            z = jnp.maximum(_hp_dot(gmean, a1_ref[...]) + c1_ref[...], 0.0)
            z2 = _hp_dot(z, a2_ref[...]) + c2_ref[...]
            aout_ref[...] = jax.nn.sigmoid(z2)

    return pl.pallas_call(
        body,
        grid=(nblk,),
        in_specs=[
            pl.BlockSpec((_BN, h_out), lambda i: (i, 0)),
            pl.BlockSpec((_BN, h_out), lambda i: (i, 0)),
            pl.BlockSpec((_BN, h_out), lambda i: (i, 0)),
            pl.BlockSpec((1, h_out), lambda i: (0, 0)),
            pl.BlockSpec((_BN, 16), lambda i: (i, 0)),
            pl.BlockSpec((1, 1, _BN), lambda i: (i, 0, 0)),
            pl.BlockSpec((h_out, mh), lambda i: (0, 0)),
            pl.BlockSpec((1, mh), lambda i: (0, 0)),
            pl.BlockSpec((mh, 1), lambda i: (0, 0)),
            pl.BlockSpec((1, 1), lambda i: (0, 0)),
        ],
        out_specs=[
            pl.BlockSpec((_BN, h_out), lambda i: (i, 0)),
            pl.BlockSpec((_G, 1), lambda i: (0, 0)),
        ],
        out_shape=[jax.ShapeDtypeStruct((n, h_out), jnp.float32),
                   jax.ShapeDtypeStruct((_G, 1), jnp.float32)],
        scratch_shapes=[pltpu.VMEM((_G, h_out), jnp.float32),
                        pltpu.VMEM((1, _G), jnp.float32)],
    )


def kernel(x, edge_index, batch, W1, b1, W2, b2, W3, b3, A1, c1, A2, c2):
    n, f_in = x.shape
    e = edge_index.shape[1]
    h = W1.shape[1]
    out = W3.shape[1]
    mh = A1.shape[1]

    src = edge_index[0]
    dst = edge_index[1]

    (t1raw,) = _tc_mm_first(n, f_in, h)(x, W1)
    deg0, deg1 = _deg_kernel(n, e)(dst)

    t1lo, t1hi, dinv = _tc_first(n, h)(t1raw, deg0, deg1)
    a1lo, a1hi = _prop_kernel(n, e, h // 2)(src, dst, t1lo, t1hi)

    t2lo, t2hi = _tc_mid(n, h, h, True)(a1lo, a1hi, t1lo, t1hi,
                                        b1.reshape(1, h), W2, dinv)
    a2lo, a2hi = _prop_kernel(n, e, h // 2)(src, dst, t2lo, t2hi)

    (t3,) = _tc_mid(n, h, out, False)(a2lo, a2hi, t2lo, t2hi,
                                      b2.reshape(1, h), W3, dinv)
    a3p0, a3p1 = _prop_edge_split(n, e, out)(src, dst, t3)

    hfinal, a = _tc_last(n, out, mh)(
        a3p0, a3p1, t3, b3.reshape(1, out), dinv,
        batch.reshape(n // _BN, 1, _BN),
        A1, c1.reshape(1, mh), A2, c2.reshape(1, 1))
    return (hfinal, a)
